# baseline (speedup 1.0000x reference)
import functools

import jax
import jax.numpy as jnp
from jax import lax
from jax.experimental import pallas as pl
from jax.experimental.pallas import tpu as pltpu
from jax.experimental.pallas import tpu_sc as plsc

N_NODES = 10000
D = 128
N_EDGES = 320000

NC = 2
NS = 16
NW = NC * NS
CH = 80
NCHUNK = 128
EPT = NCHUNK * CH
E_PAD = NW * EPT
ACC_PAD = 10240
EPT_REAL = N_EDGES // NW
T_PAD = 10016
RPT = ACC_PAD // NS
DEG_PAD = 10240
DEG_PT = DEG_PAD // NS
NBUF = 4
PCH = NCHUNK // 4

_MESH = plsc.VectorSubcoreMesh(core_axis_name="c", subcore_axis_name="s")

ROW_BLK = 512
GRID = pl.cdiv(N_NODES, ROW_BLK)


@functools.partial(
    pl.kernel,
    out_type=jax.ShapeDtypeStruct((NC * DEG_PAD,), jnp.float32),
    mesh=_MESH,
    scratch_types=[
        pltpu.VMEM((NCHUNK, CH), jnp.int32),
        pltpu.VMEM((CH,), jnp.float32),
        pltpu.VMEM((DEG_PT,), jnp.float32),
        pltpu.VMEM_SHARED((DEG_PAD,), jnp.float32),
        pltpu.SemaphoreType.DMA,
    ],
)
def _deg_kernel(dst_hbm, out_hbm, idx_v, ones_v, zbuf_v, deg_s, sem):
    cid = lax.axis_index("c")
    sid = lax.axis_index("s")
    wid = cid * NS + sid

    def fill_ones(i, _):
        ones_v[pl.ds(i * 16, 16)] = jnp.ones((16,), jnp.float32)
        return 0

    lax.fori_loop(0, CH // 16, fill_ones, 0)

    def fill_zeros(i, _):
        zbuf_v[pl.ds(i * 16, 16)] = jnp.zeros((16,), jnp.float32)
        return 0

    lax.fori_loop(0, DEG_PT // 16, fill_zeros, 0)
    pltpu.sync_copy(zbuf_v, deg_s.at[pl.ds(sid * DEG_PT, DEG_PT)])
    pltpu.sync_copy(dst_hbm.at[wid], idx_v)
    plsc.subcore_barrier()

    GRP = 8

    def body(g, _):
        for b in range(GRP):
            pltpu.async_copy(ones_v, deg_s.at[idx_v.at[g * GRP + b]], sem,
                             add=True)
        for b in range(GRP):
            pltpu.make_async_copy(ones_v, deg_s.at[idx_v.at[g * GRP + b]],
                                  sem).wait()
        return 0

    lax.fori_loop(0, NCHUNK // GRP, body, 0)
    plsc.subcore_barrier()
    pltpu.sync_copy(
        deg_s.at[pl.ds(sid * DEG_PT, DEG_PT)],
        out_hbm.at[pl.ds(cid * DEG_PAD + sid * DEG_PT, DEG_PT)],
    )


@functools.partial(
    pl.kernel,
    out_type=jax.ShapeDtypeStruct((NC, ACC_PAD, D), jnp.float32),
    mesh=_MESH,
    scratch_types=[
        pltpu.VMEM((PCH, CH), jnp.int32),
        pltpu.VMEM((PCH, CH), jnp.int32),
        [pltpu.VMEM((CH, D), jnp.float32)] * NBUF,
        pltpu.VMEM_SHARED((ACC_PAD, D), jnp.float32),
        [pltpu.SemaphoreType.DMA] * NBUF,
        [pltpu.SemaphoreType.DMA] * NBUF,
    ],
)
def _edge_kernel(t_hbm, src_hbm, dst_hbm, out_hbm,
                 src_v, dst_v, rows, acc_s, gsem, ssem):
    cid = lax.axis_index("c")
    sid = lax.axis_index("s")
    wid = cid * NS + sid

    def fill_zeros(k, _):
        r = k // (D // 16)
        c = (k % (D // 16)) * 16
        rows[0][r, pl.ds(c, 16)] = jnp.zeros((16,), jnp.float32)
        return 0

    lax.fori_loop(0, CH * (D // 16), fill_zeros, 0)
    for z in range(RPT // CH):
        pltpu.async_copy(rows[0], acc_s.at[pl.ds(sid * RPT + z * CH, CH)],
                         gsem[0])
    for z in range(RPT // CH):
        pltpu.make_async_copy(rows[0], acc_s.at[pl.ds(sid * RPT, CH)],
                              gsem[0]).wait()
    plsc.subcore_barrier()

    def gather(c, b):
        pltpu.async_copy(t_hbm.at[src_v.at[c]], rows[b], gsem[b])

    def gather_wait(c, b):
        pltpu.make_async_copy(t_hbm.at[src_v.at[c]], rows[b], gsem[b]).wait()

    def scatter(c, b):
        pltpu.async_copy(rows[b], acc_s.at[dst_v.at[c]], ssem[b], add=True)

    def scatter_wait(c, b):
        pltpu.make_async_copy(rows[b], acc_s.at[dst_v.at[c]], ssem[b]).wait()

    for p in range(NCHUNK // PCH):
        pltpu.sync_copy(src_hbm.at[wid, pl.ds(p * PCH, PCH)], src_v)
        pltpu.sync_copy(dst_hbm.at[wid, pl.ds(p * PCH, PCH)], dst_v)
        for b in range(NBUF):
            gather(b, b)

        def body(h, _):
            c0 = h * NBUF
            for b in range(NBUF):
                gather_wait(c0 + b, b)
                scatter(c0 + b, b)
            for b in range(NBUF):
                scatter_wait(c0 + b, b)
                gather(c0 + NBUF + b, b)
            return 0

        lax.fori_loop(0, PCH // NBUF - 1, body, 0)
        c0 = PCH - NBUF
        for b in range(NBUF):
            gather_wait(c0 + b, b)
            scatter(c0 + b, b)
        for b in range(NBUF):
            scatter_wait(c0 + b, b)
    plsc.subcore_barrier()
    pltpu.sync_copy(
        acc_s.at[pl.ds(sid * RPT, RPT)],
        out_hbm.at[cid, pl.ds(sid * RPT, RPT)],
    )


def _dinv_block(degp_ref, i):
    off = pl.multiple_of(i * ROW_BLK, 128)
    deg = (degp_ref[0, pl.ds(off, ROW_BLK)]
           + degp_ref[1, pl.ds(off, ROW_BLK)] + 1.0)
    return lax.rsqrt(deg)


def _row_mask(i, v):
    r = i * ROW_BLK + lax.broadcasted_iota(jnp.int32, (ROW_BLK, 1), 0)
    return jnp.where(r < N_NODES, v, 0.0)


def _stage_a_body(degp_ref, x_ref, w_ref, out_ref):
    i = pl.program_id(0)
    dinv = _dinv_block(degp_ref, i)
    xw = jnp.dot(x_ref[...], w_ref[...], preferred_element_type=jnp.float32)
    out_ref[...] = _row_mask(i, dinv[:, None] * xw)


def _stage_b_body(degp_ref, acc_ref, t_ref, b_ref, w_ref, out_ref):
    i = pl.program_id(0)
    dinv = _dinv_block(degp_ref, i)
    s = acc_ref[0] + acc_ref[1] + t_ref[...]
    h = jnp.maximum(dinv[:, None] * s + b_ref[...], 0.0)
    out_ref[...] = _row_mask(i, dinv[:, None] * jnp.dot(
        h, w_ref[...], preferred_element_type=jnp.float32))


def _stage_c_body(degp_ref, acc_ref, t_ref, b_ref, out_ref):
    i = pl.program_id(0)
    dinv = _dinv_block(degp_ref, i)
    s = acc_ref[0] + acc_ref[1] + t_ref[...]
    out_ref[...] = dinv[:, None] * s + b_ref[...]


_DEGP_SPEC = pl.BlockSpec((NC, DEG_PAD), lambda i: (0, 0))
_ROW_SPEC = pl.BlockSpec((ROW_BLK, D), lambda i: (i, 0))
_ACC_SPEC = pl.BlockSpec((NC, ROW_BLK, D), lambda i: (0, i, 0))
_MAT_SPEC = pl.BlockSpec((D, D), lambda i: (0, 0))
_BIAS_SPEC = pl.BlockSpec((1, D), lambda i: (0, 0))
_OUT_SHAPE = jax.ShapeDtypeStruct((N_NODES, D), jnp.float32)
_OUT_SHAPE_PAD = jax.ShapeDtypeStruct((T_PAD, D), jnp.float32)

_stage_a = pl.pallas_call(
    _stage_a_body,
    grid=(GRID,),
    in_specs=[_DEGP_SPEC, _ROW_SPEC, _MAT_SPEC],
    out_specs=_ROW_SPEC,
    out_shape=_OUT_SHAPE_PAD,
)

_stage_b = pl.pallas_call(
    _stage_b_body,
    grid=(GRID,),
    in_specs=[_DEGP_SPEC, _ACC_SPEC, _ROW_SPEC, _BIAS_SPEC, _MAT_SPEC],
    out_specs=_ROW_SPEC,
    out_shape=_OUT_SHAPE_PAD,
)

_stage_c = pl.pallas_call(
    _stage_c_body,
    grid=(GRID,),
    in_specs=[_DEGP_SPEC, _ACC_SPEC, _ROW_SPEC, _BIAS_SPEC],
    out_specs=_ROW_SPEC,
    out_shape=_OUT_SHAPE,
)


def kernel(x, edge_index, W1, b1, W2, b2):
    ei = edge_index.astype(jnp.int32)
    npad = EPT - EPT_REAL
    j = jnp.arange(npad, dtype=jnp.int32)
    src = jnp.concatenate(
        [ei[0].reshape(NW, EPT_REAL),
         jnp.broadcast_to(N_NODES + (j % 16), (NW, npad))],
        axis=1).reshape(NW, NCHUNK, CH)
    wids = jnp.arange(NW, dtype=jnp.int32)[:, None]
    dst_edge = jnp.concatenate(
        [ei[1].reshape(NW, EPT_REAL), (wids * npad + j) % ACC_PAD],
        axis=1).reshape(NW, NCHUNK, CH)
    dst_deg = jnp.concatenate(
        [ei[1].reshape(NW, EPT_REAL),
         jnp.broadcast_to(N_NODES + (j % (DEG_PAD - N_NODES)), (NW, npad))],
        axis=1).reshape(NW, NCHUNK, CH)
    degp = _deg_kernel(dst_deg).reshape(NC, DEG_PAD)
    t1 = _stage_a(degp, x, W1)
    acc1 = _edge_kernel(t1, src, dst_edge)
    t2 = _stage_b(degp, acc1, t1, b1.reshape(1, D), W2)
    acc2 = _edge_kernel(t2, src, dst_edge)
    return _stage_c(degp, acc2, t2, b2.reshape(1, D))


# speedup vs baseline: 1.0465x; 1.0465x over previous
import functools

import jax
import jax.numpy as jnp
from jax import lax
from jax.experimental import pallas as pl
from jax.experimental.pallas import tpu as pltpu
from jax.experimental.pallas import tpu_sc as plsc

N_NODES = 10000
D = 128
N_EDGES = 320000

NC = 2
NS = 16
NW = NC * NS
CH = 80
NCHUNK = 128
EPT = NCHUNK * CH
E_PAD = NW * EPT
ACC_PAD = 10240
EPT_REAL = N_EDGES // NW
T_PAD = 10016
RPT = ACC_PAD // NS
DEG_PAD = 10240
DEG_PT = DEG_PAD // NS
NBUF = 4
PCH = NCHUNK // 4

_MESH = plsc.VectorSubcoreMesh(core_axis_name="c", subcore_axis_name="s")

ROW_BLK = 1024
GRID = pl.cdiv(N_NODES, ROW_BLK)


@functools.partial(
    pl.kernel,
    out_type=jax.ShapeDtypeStruct((NC * DEG_PAD,), jnp.float32),
    mesh=_MESH,
    scratch_types=[
        pltpu.VMEM((NCHUNK, CH), jnp.int32),
        pltpu.VMEM((CH,), jnp.float32),
        pltpu.VMEM((DEG_PT,), jnp.float32),
        pltpu.VMEM_SHARED((DEG_PAD,), jnp.float32),
        pltpu.SemaphoreType.DMA,
    ],
)
def _deg_kernel(dst_hbm, out_hbm, idx_v, ones_v, zbuf_v, deg_s, sem):
    cid = lax.axis_index("c")
    sid = lax.axis_index("s")
    wid = cid * NS + sid

    def fill_ones(i, _):
        ones_v[pl.ds(i * 16, 16)] = jnp.ones((16,), jnp.float32)
        return 0

    lax.fori_loop(0, CH // 16, fill_ones, 0)

    def fill_zeros(i, _):
        zbuf_v[pl.ds(i * 16, 16)] = jnp.zeros((16,), jnp.float32)
        return 0

    lax.fori_loop(0, DEG_PT // 16, fill_zeros, 0)
    pltpu.sync_copy(zbuf_v, deg_s.at[pl.ds(sid * DEG_PT, DEG_PT)])
    pltpu.sync_copy(dst_hbm.at[wid], idx_v)
    plsc.subcore_barrier()

    GRP = 8

    def body(g, _):
        for b in range(GRP):
            pltpu.async_copy(ones_v, deg_s.at[idx_v.at[g * GRP + b]], sem,
                             add=True)
        for b in range(GRP):
            pltpu.make_async_copy(ones_v, deg_s.at[idx_v.at[g * GRP + b]],
                                  sem).wait()
        return 0

    lax.fori_loop(0, NCHUNK // GRP, body, 0)
    plsc.subcore_barrier()
    pltpu.sync_copy(
        deg_s.at[pl.ds(sid * DEG_PT, DEG_PT)],
        out_hbm.at[pl.ds(cid * DEG_PAD + sid * DEG_PT, DEG_PT)],
    )


@functools.partial(
    pl.kernel,
    out_type=jax.ShapeDtypeStruct((NC, ACC_PAD, D), jnp.float32),
    mesh=_MESH,
    scratch_types=[
        pltpu.VMEM((PCH, CH), jnp.int32),
        pltpu.VMEM((PCH, CH), jnp.int32),
        [pltpu.VMEM((CH, D), jnp.float32)] * NBUF,
        pltpu.VMEM_SHARED((ACC_PAD, D), jnp.float32),
        [pltpu.SemaphoreType.DMA] * NBUF,
        [pltpu.SemaphoreType.DMA] * NBUF,
    ],
)
def _edge_kernel(t_hbm, src_hbm, dst_hbm, out_hbm,
                 src_v, dst_v, rows, acc_s, gsem, ssem):
    cid = lax.axis_index("c")
    sid = lax.axis_index("s")
    wid = cid * NS + sid

    def fill_zeros(k, _):
        r = k // (D // 16)
        c = (k % (D // 16)) * 16
        rows[0][r, pl.ds(c, 16)] = jnp.zeros((16,), jnp.float32)
        return 0

    lax.fori_loop(0, CH * (D // 16), fill_zeros, 0)
    for z in range(RPT // CH):
        pltpu.async_copy(rows[0], acc_s.at[pl.ds(sid * RPT + z * CH, CH)],
                         gsem[0])
    for z in range(RPT // CH):
        pltpu.make_async_copy(rows[0], acc_s.at[pl.ds(sid * RPT, CH)],
                              gsem[0]).wait()
    plsc.subcore_barrier()

    def gather(c, b):
        pltpu.async_copy(t_hbm.at[src_v.at[c]], rows[b], gsem[b])

    def gather_wait(c, b):
        pltpu.make_async_copy(t_hbm.at[src_v.at[c]], rows[b], gsem[b]).wait()

    def scatter(c, b):
        pltpu.async_copy(rows[b], acc_s.at[dst_v.at[c]], ssem[b], add=True)

    def scatter_wait(c, b):
        pltpu.make_async_copy(rows[b], acc_s.at[dst_v.at[c]], ssem[b]).wait()

    for p in range(NCHUNK // PCH):
        pltpu.sync_copy(src_hbm.at[wid, pl.ds(p * PCH, PCH)], src_v)
        pltpu.sync_copy(dst_hbm.at[wid, pl.ds(p * PCH, PCH)], dst_v)
        for b in range(NBUF):
            gather(b, b)

        def body(h, _):
            c0 = h * NBUF
            for b in range(NBUF):
                gather_wait(c0 + b, b)
                scatter(c0 + b, b)
            for b in range(NBUF):
                scatter_wait(c0 + b, b)
                gather(c0 + NBUF + b, b)
            return 0

        lax.fori_loop(0, PCH // NBUF - 1, body, 0)
        c0 = PCH - NBUF
        for b in range(NBUF):
            gather_wait(c0 + b, b)
            scatter(c0 + b, b)
        for b in range(NBUF):
            scatter_wait(c0 + b, b)
    plsc.subcore_barrier()
    pltpu.sync_copy(
        acc_s.at[pl.ds(sid * RPT, RPT)],
        out_hbm.at[cid, pl.ds(sid * RPT, RPT)],
    )


def _dinv_block(degp_ref, i):
    off = pl.multiple_of(i * ROW_BLK, 128)
    deg = (degp_ref[0, pl.ds(off, ROW_BLK)]
           + degp_ref[1, pl.ds(off, ROW_BLK)] + 1.0)
    return lax.rsqrt(deg)


def _row_mask(i, v):
    r = i * ROW_BLK + lax.broadcasted_iota(jnp.int32, (ROW_BLK, 1), 0)
    return jnp.where(r < N_NODES, v, 0.0)


def _stage_a_body(degp_ref, x_ref, w_ref, out_ref):
    i = pl.program_id(0)
    dinv = _dinv_block(degp_ref, i)
    xw = jnp.dot(x_ref[...], w_ref[...], preferred_element_type=jnp.float32)
    out_ref[...] = _row_mask(i, dinv[:, None] * xw)


def _stage_b_body(degp_ref, acc_ref, t_ref, b_ref, w_ref, out_ref):
    i = pl.program_id(0)
    dinv = _dinv_block(degp_ref, i)
    s = acc_ref[0] + acc_ref[1] + t_ref[...]
    h = jnp.maximum(dinv[:, None] * s + b_ref[...], 0.0)
    out_ref[...] = _row_mask(i, dinv[:, None] * jnp.dot(
        h, w_ref[...], preferred_element_type=jnp.float32))


def _stage_c_body(degp_ref, acc_ref, t_ref, b_ref, out_ref):
    i = pl.program_id(0)
    dinv = _dinv_block(degp_ref, i)
    s = acc_ref[0] + acc_ref[1] + t_ref[...]
    out_ref[...] = dinv[:, None] * s + b_ref[...]


_DEGP_SPEC = pl.BlockSpec((NC, DEG_PAD), lambda i: (0, 0))
_ROW_SPEC = pl.BlockSpec((ROW_BLK, D), lambda i: (i, 0))
_ACC_SPEC = pl.BlockSpec((NC, ROW_BLK, D), lambda i: (0, i, 0))
_MAT_SPEC = pl.BlockSpec((D, D), lambda i: (0, 0))
_BIAS_SPEC = pl.BlockSpec((1, D), lambda i: (0, 0))
_OUT_SHAPE = jax.ShapeDtypeStruct((N_NODES, D), jnp.float32)
_OUT_SHAPE_PAD = jax.ShapeDtypeStruct((T_PAD, D), jnp.float32)

_stage_a = pl.pallas_call(
    _stage_a_body,
    grid=(GRID,),
    in_specs=[_DEGP_SPEC, _ROW_SPEC, _MAT_SPEC],
    out_specs=_ROW_SPEC,
    out_shape=_OUT_SHAPE_PAD,
)

_stage_b = pl.pallas_call(
    _stage_b_body,
    grid=(GRID,),
    in_specs=[_DEGP_SPEC, _ACC_SPEC, _ROW_SPEC, _BIAS_SPEC, _MAT_SPEC],
    out_specs=_ROW_SPEC,
    out_shape=_OUT_SHAPE_PAD,
)

_stage_c = pl.pallas_call(
    _stage_c_body,
    grid=(GRID,),
    in_specs=[_DEGP_SPEC, _ACC_SPEC, _ROW_SPEC, _BIAS_SPEC],
    out_specs=_ROW_SPEC,
    out_shape=_OUT_SHAPE,
)


def kernel(x, edge_index, W1, b1, W2, b2):
    ei = edge_index.astype(jnp.int32)
    npad = EPT - EPT_REAL
    j = jnp.arange(npad, dtype=jnp.int32)
    src = jnp.concatenate(
        [ei[0].reshape(NW, EPT_REAL),
         jnp.broadcast_to(N_NODES + (j % 16), (NW, npad))],
        axis=1).reshape(NW, NCHUNK, CH)
    wids = jnp.arange(NW, dtype=jnp.int32)[:, None]
    dst_edge = jnp.concatenate(
        [ei[1].reshape(NW, EPT_REAL), (wids * npad + j) % ACC_PAD],
        axis=1).reshape(NW, NCHUNK, CH)
    dst_deg = jnp.concatenate(
        [ei[1].reshape(NW, EPT_REAL),
         jnp.broadcast_to(N_NODES + (j % (DEG_PAD - N_NODES)), (NW, npad))],
        axis=1).reshape(NW, NCHUNK, CH)
    degp = _deg_kernel(dst_deg).reshape(NC, DEG_PAD)
    t1 = _stage_a(degp, x, W1)
    acc1 = _edge_kernel(t1, src, dst_edge)
    t2 = _stage_b(degp, acc1, t1, b1.reshape(1, D), W2)
    acc2 = _edge_kernel(t2, src, dst_edge)
    return _stage_c(degp, acc2, t2, b2.reshape(1, D))


# speedup vs baseline: 1.0645x; 1.0172x over previous
import functools

import jax
import jax.numpy as jnp
from jax import lax
from jax.experimental import pallas as pl
from jax.experimental.pallas import tpu as pltpu
from jax.experimental.pallas import tpu_sc as plsc

N_NODES = 10000
D = 128
N_EDGES = 320000

NC = 2
NS = 16
NW = NC * NS
CH = 80
NCHUNK = 128
EPT = NCHUNK * CH
E_PAD = NW * EPT
ACC_PAD = 10240
EPT_REAL = N_EDGES // NW
T_PAD = 10016
RPT = ACC_PAD // NS
DEG_PAD = 10240
DEG_PT = DEG_PAD // NS
NBUF = 4
PCH = NCHUNK // 4

_MESH = plsc.VectorSubcoreMesh(core_axis_name="c", subcore_axis_name="s")

ROW_BLK = 2048
GRID = pl.cdiv(N_NODES, ROW_BLK)


@functools.partial(
    pl.kernel,
    out_type=jax.ShapeDtypeStruct((NC * DEG_PAD,), jnp.float32),
    mesh=_MESH,
    scratch_types=[
        pltpu.VMEM((NCHUNK, CH), jnp.int32),
        pltpu.VMEM((CH,), jnp.float32),
        pltpu.VMEM((DEG_PT,), jnp.float32),
        pltpu.VMEM_SHARED((DEG_PAD,), jnp.float32),
        pltpu.SemaphoreType.DMA,
    ],
)
def _deg_kernel(dst_hbm, out_hbm, idx_v, ones_v, zbuf_v, deg_s, sem):
    cid = lax.axis_index("c")
    sid = lax.axis_index("s")
    wid = cid * NS + sid

    def fill_ones(i, _):
        ones_v[pl.ds(i * 16, 16)] = jnp.ones((16,), jnp.float32)
        return 0

    lax.fori_loop(0, CH // 16, fill_ones, 0)

    def fill_zeros(i, _):
        zbuf_v[pl.ds(i * 16, 16)] = jnp.zeros((16,), jnp.float32)
        return 0

    lax.fori_loop(0, DEG_PT // 16, fill_zeros, 0)
    pltpu.sync_copy(zbuf_v, deg_s.at[pl.ds(sid * DEG_PT, DEG_PT)])
    pltpu.sync_copy(dst_hbm.at[wid], idx_v)
    plsc.subcore_barrier()

    GRP = 8

    def body(g, _):
        for b in range(GRP):
            pltpu.async_copy(ones_v, deg_s.at[idx_v.at[g * GRP + b]], sem,
                             add=True)
        for b in range(GRP):
            pltpu.make_async_copy(ones_v, deg_s.at[idx_v.at[g * GRP + b]],
                                  sem).wait()
        return 0

    lax.fori_loop(0, NCHUNK // GRP, body, 0)
    plsc.subcore_barrier()
    pltpu.sync_copy(
        deg_s.at[pl.ds(sid * DEG_PT, DEG_PT)],
        out_hbm.at[pl.ds(cid * DEG_PAD + sid * DEG_PT, DEG_PT)],
    )


@functools.partial(
    pl.kernel,
    out_type=jax.ShapeDtypeStruct((NC, ACC_PAD, D), jnp.float32),
    mesh=_MESH,
    scratch_types=[
        pltpu.VMEM((PCH, CH), jnp.int32),
        pltpu.VMEM((PCH, CH), jnp.int32),
        [pltpu.VMEM((CH, D), jnp.float32)] * NBUF,
        pltpu.VMEM_SHARED((ACC_PAD, D), jnp.float32),
        [pltpu.SemaphoreType.DMA] * NBUF,
        [pltpu.SemaphoreType.DMA] * NBUF,
    ],
)
def _edge_kernel(t_hbm, src_hbm, dst_hbm, out_hbm,
                 src_v, dst_v, rows, acc_s, gsem, ssem):
    cid = lax.axis_index("c")
    sid = lax.axis_index("s")
    wid = cid * NS + sid

    def fill_zeros(k, _):
        r = k // (D // 16)
        c = (k % (D // 16)) * 16
        rows[0][r, pl.ds(c, 16)] = jnp.zeros((16,), jnp.float32)
        return 0

    lax.fori_loop(0, CH * (D // 16), fill_zeros, 0)
    for z in range(RPT // CH):
        pltpu.async_copy(rows[0], acc_s.at[pl.ds(sid * RPT + z * CH, CH)],
                         gsem[0])
    for z in range(RPT // CH):
        pltpu.make_async_copy(rows[0], acc_s.at[pl.ds(sid * RPT, CH)],
                              gsem[0]).wait()
    plsc.subcore_barrier()

    def gather(c, b):
        pltpu.async_copy(t_hbm.at[src_v.at[c]], rows[b], gsem[b])

    def gather_wait(c, b):
        pltpu.make_async_copy(t_hbm.at[src_v.at[c]], rows[b], gsem[b]).wait()

    def scatter(c, b):
        pltpu.async_copy(rows[b], acc_s.at[dst_v.at[c]], ssem[b], add=True)

    def scatter_wait(c, b):
        pltpu.make_async_copy(rows[b], acc_s.at[dst_v.at[c]], ssem[b]).wait()

    for p in range(NCHUNK // PCH):
        pltpu.sync_copy(src_hbm.at[wid, pl.ds(p * PCH, PCH)], src_v)
        pltpu.sync_copy(dst_hbm.at[wid, pl.ds(p * PCH, PCH)], dst_v)
        for b in range(NBUF):
            gather(b, b)

        def body(h, _):
            c0 = h * NBUF
            for b in range(NBUF):
                gather_wait(c0 + b, b)
                scatter(c0 + b, b)
            for b in range(NBUF):
                scatter_wait(c0 + b, b)
                gather(c0 + NBUF + b, b)
            return 0

        lax.fori_loop(0, PCH // NBUF - 1, body, 0)
        c0 = PCH - NBUF
        for b in range(NBUF):
            gather_wait(c0 + b, b)
            scatter(c0 + b, b)
        for b in range(NBUF):
            scatter_wait(c0 + b, b)
    plsc.subcore_barrier()
    pltpu.sync_copy(
        acc_s.at[pl.ds(sid * RPT, RPT)],
        out_hbm.at[cid, pl.ds(sid * RPT, RPT)],
    )


def _dinv_block(degp_ref, i):
    off = pl.multiple_of(i * ROW_BLK, 128)
    deg = (degp_ref[0, pl.ds(off, ROW_BLK)]
           + degp_ref[1, pl.ds(off, ROW_BLK)] + 1.0)
    return lax.rsqrt(deg)


def _row_mask(i, v):
    r = i * ROW_BLK + lax.broadcasted_iota(jnp.int32, (ROW_BLK, 1), 0)
    return jnp.where(r < N_NODES, v, 0.0)


def _stage_a_body(degp_ref, x_ref, w_ref, out_ref):
    i = pl.program_id(0)
    dinv = _dinv_block(degp_ref, i)
    xw = jnp.dot(x_ref[...], w_ref[...], preferred_element_type=jnp.float32)
    out_ref[...] = _row_mask(i, dinv[:, None] * xw)


def _stage_b_body(degp_ref, acc_ref, t_ref, b_ref, w_ref, out_ref):
    i = pl.program_id(0)
    dinv = _dinv_block(degp_ref, i)
    s = acc_ref[0] + acc_ref[1] + t_ref[...]
    h = jnp.maximum(dinv[:, None] * s + b_ref[...], 0.0)
    out_ref[...] = _row_mask(i, dinv[:, None] * jnp.dot(
        h, w_ref[...], preferred_element_type=jnp.float32))


def _stage_c_body(degp_ref, acc_ref, t_ref, b_ref, out_ref):
    i = pl.program_id(0)
    dinv = _dinv_block(degp_ref, i)
    s = acc_ref[0] + acc_ref[1] + t_ref[...]
    out_ref[...] = dinv[:, None] * s + b_ref[...]


_DEGP_SPEC = pl.BlockSpec((NC, DEG_PAD), lambda i: (0, 0))
_ROW_SPEC = pl.BlockSpec((ROW_BLK, D), lambda i: (i, 0))
_ACC_SPEC = pl.BlockSpec((NC, ROW_BLK, D), lambda i: (0, i, 0))
_MAT_SPEC = pl.BlockSpec((D, D), lambda i: (0, 0))
_BIAS_SPEC = pl.BlockSpec((1, D), lambda i: (0, 0))
_OUT_SHAPE = jax.ShapeDtypeStruct((N_NODES, D), jnp.float32)
_OUT_SHAPE_PAD = jax.ShapeDtypeStruct((T_PAD, D), jnp.float32)

_stage_a = pl.pallas_call(
    _stage_a_body,
    grid=(GRID,),
    in_specs=[_DEGP_SPEC, _ROW_SPEC, _MAT_SPEC],
    out_specs=_ROW_SPEC,
    out_shape=_OUT_SHAPE_PAD,
)

_stage_b = pl.pallas_call(
    _stage_b_body,
    grid=(GRID,),
    in_specs=[_DEGP_SPEC, _ACC_SPEC, _ROW_SPEC, _BIAS_SPEC, _MAT_SPEC],
    out_specs=_ROW_SPEC,
    out_shape=_OUT_SHAPE_PAD,
)

_stage_c = pl.pallas_call(
    _stage_c_body,
    grid=(GRID,),
    in_specs=[_DEGP_SPEC, _ACC_SPEC, _ROW_SPEC, _BIAS_SPEC],
    out_specs=_ROW_SPEC,
    out_shape=_OUT_SHAPE,
)


def kernel(x, edge_index, W1, b1, W2, b2):
    ei = edge_index.astype(jnp.int32)
    npad = EPT - EPT_REAL
    j = jnp.arange(npad, dtype=jnp.int32)
    src = jnp.concatenate(
        [ei[0].reshape(NW, EPT_REAL),
         jnp.broadcast_to(N_NODES + (j % 16), (NW, npad))],
        axis=1).reshape(NW, NCHUNK, CH)
    wids = jnp.arange(NW, dtype=jnp.int32)[:, None]
    dst_edge = jnp.concatenate(
        [ei[1].reshape(NW, EPT_REAL), (wids * npad + j) % ACC_PAD],
        axis=1).reshape(NW, NCHUNK, CH)
    dst_deg = jnp.concatenate(
        [ei[1].reshape(NW, EPT_REAL),
         jnp.broadcast_to(N_NODES + (j % (DEG_PAD - N_NODES)), (NW, npad))],
        axis=1).reshape(NW, NCHUNK, CH)
    degp = _deg_kernel(dst_deg).reshape(NC, DEG_PAD)
    t1 = _stage_a(degp, x, W1)
    acc1 = _edge_kernel(t1, src, dst_edge)
    t2 = _stage_b(degp, acc1, t1, b1.reshape(1, D), W2)
    acc2 = _edge_kernel(t2, src, dst_edge)
    return _stage_c(degp, acc2, t2, b2.reshape(1, D))
